# Initial kernel scaffold; baseline (speedup 1.0000x reference)
import jax
import jax.numpy as jnp
from jax.experimental import pallas as pl


def kernel(x, p_x, w1, b1, g1, w2, b2, g2, w3, b3, g3, w4, b4, g4, wenc, benc, wdec, bdec):
    raise NotImplementedError("write your pallas kernel here")


# speedup vs baseline: 1.8048x; 1.8048x over previous
import functools

import jax
import jax.numpy as jnp
from jax.experimental import pallas as pl
from jax.experimental.pallas import tpu as pltpu

_NEG_SLOPE = 0.01


def _lrelu(v):
    return jnp.where(v > 0, v, _NEG_SLOPE * v)


def _planes(x):
    tb, h, w, c = x.shape
    xr = x.reshape(tb, h // 2, 2, w, c)
    xe, xo = xr[:, :, 0], xr[:, :, 1]

    def cols(a):
        ar = a.reshape(tb, h // 2, w // 2, 2, c)
        return ar[:, :, :, 0], ar[:, :, :, 1]

    xee, xeo = cols(xe)
    xoe, xoo = cols(xo)
    return xee, xeo, xoe, xoo


def _shift_r(a):
    return jnp.concatenate([jnp.zeros_like(a[:, :1]), a[:, :-1]], axis=1)


def _shift_c(a):
    return jnp.concatenate([jnp.zeros_like(a[:, :, :1]), a[:, :, :-1]], axis=2)


def _conv_s2(x, wcol, bmap):
    tb, h, w, c = x.shape
    ho, wo = h // 2, w // 2
    xee, xeo, xoe, xoo = _planes(x)
    taps = [
        _shift_r(_shift_c(xoo)), _shift_r(xoe), _shift_r(xoo),
        _shift_c(xeo),           xee,           xeo,
        _shift_c(xoo),           xoe,           xoo,
    ]
    t = jnp.concatenate(taps, axis=3).reshape(tb * ho * wo, 9 * c)
    y = jnp.dot(t, wcol, preferred_element_type=jnp.float32)
    return y.reshape(tb, ho, wo, wcol.shape[1]) + bmap[None]


def _deconv_s2(d, wee, weo, woe, woo, bmap):
    tb, h, w, c = d.shape
    cout = wee.shape[1]
    dr = jnp.concatenate([d[:, 1:], jnp.zeros_like(d[:, :1])], axis=1)
    dc = jnp.concatenate([d[:, :, 1:], jnp.zeros_like(d[:, :, :1])], axis=2)
    drc = jnp.concatenate([dr[:, :, 1:], jnp.zeros_like(dr[:, :, :1])], axis=2)

    def mm(t, wc):
        k = t.shape[-1]
        y = jnp.dot(t.reshape(tb * h * w, k), wc,
                    preferred_element_type=jnp.float32)
        return y.reshape(tb, h, w, cout)

    y_ee = mm(d, wee)
    y_eo = mm(jnp.concatenate([dc, d], axis=3), weo)
    y_oe = mm(jnp.concatenate([dr, d], axis=3), woe)
    y_oo = mm(jnp.concatenate([drc, dr, dc, d], axis=3), woo)

    ye = jnp.stack([y_ee, y_eo], axis=3).reshape(tb, h, 2 * w, cout)
    yo = jnp.stack([y_oe, y_oo], axis=3).reshape(tb, h, 2 * w, cout)
    y = jnp.stack([ye, yo], axis=2).reshape(tb, 2 * h, 2 * w, cout)
    return y + bmap[None]


def _enc_kernel(x_ref, px_ref, w1_ref, b1_ref, w2_ref, b2_ref,
                wenc_ref, benc_ref, wdec_ref, bdec_ref,
                z_ref, d_ref, *, n_keep):
    tb = x_ref.shape[0]
    x = jnp.transpose(x_ref[...], (0, 2, 3, 1))
    h1 = _lrelu(_conv_s2(x, w1_ref[...], b1_ref[...]))
    h2 = _conv_s2(h1, w2_ref[...], b2_ref[...])
    e = h2.reshape(tb, -1)
    z0 = jnp.dot(e, wenc_ref[...], preferred_element_type=jnp.float32)
    z0 = z0 + benc_ref[...]
    z = jnp.concatenate([z0[:, :n_keep], px_ref[...]], axis=1)
    z_ref[...] = z
    d = jnp.dot(z, wdec_ref[...], preferred_element_type=jnp.float32)
    d_ref[...] = d + bdec_ref[...]


def _dec_kernel(d_ref,
                w3ee_ref, w3eo_ref, w3oe_ref, w3oo_ref, b3_ref,
                w4ee_ref, w4eo_ref, w4oe_ref, w4oo_ref, b4_ref,
                out_ref):
    d = d_ref[...]
    u1 = _lrelu(_deconv_s2(d, w3ee_ref[...], w3eo_ref[...], w3oe_ref[...],
                           w3oo_ref[...], b3_ref[...]))
    u2 = _deconv_s2(u1, w4ee_ref[...], w4eo_ref[...], w4oe_ref[...],
                    w4oo_ref[...], b4_ref[...])
    out_ref[...] = jnp.transpose(u2, (0, 3, 1, 2))


def _conv_wcol(w):
    t, cout, cin = w.shape
    return jnp.transpose(w, (0, 2, 1)).reshape(t * cin, cout)


def _deconv_plane_w(w):
    wt = jnp.transpose(w, (0, 2, 1))
    wee = wt[4]
    weo = jnp.concatenate([wt[3], wt[5]], axis=0)
    woe = jnp.concatenate([wt[1], wt[7]], axis=0)
    woo = jnp.concatenate([wt[0], wt[2], wt[6], wt[8]], axis=0)
    return wee, weo, woe, woo


def _bias_map(b, ho, wo):
    return jnp.transpose(b, (1, 0)).reshape(ho, wo, b.shape[0])


def kernel(x, p_x, w1, b1, g1, w2, b2, g2, w3, b3, g3, w4, b4, g4,
           wenc, benc, wdec, bdec):
    del g1, g2, g3, g4
    bsz, cin, h, w = x.shape
    h1, w1s = h // 2, w // 2
    h2, w2s = h1 // 2, w1s // 2
    c1 = w1.shape[1]
    c2 = w2.shape[1]
    latent = wenc.shape[1]
    p = p_x.shape[1]

    w1c = _conv_wcol(w1)
    w2c = _conv_wcol(w2)
    w3ee, w3eo, w3oe, w3oo = _deconv_plane_w(w3)
    w4ee, w4eo, w4oe, w4oo = _deconv_plane_w(w4)
    b1m = _bias_map(b1, h1, w1s)
    b2m = _bias_map(b2, h2, w2s)
    b3m = _bias_map(b3, h1, w1s)
    b4m = _bias_map(b4, h, w)
    wenc_h = wenc.reshape(c2, h2, w2s, latent).transpose(1, 2, 0, 3)
    wenc_h = wenc_h.reshape(c2 * h2 * w2s, latent)
    wdec_h = wdec.reshape(latent, c2, h2, w2s).transpose(0, 2, 3, 1)
    wdec_h = wdec_h.reshape(latent, c2 * h2 * w2s)
    bdec_h = bdec.reshape(c2, h2, w2s).transpose(1, 2, 0).reshape(1, -1)

    tbatch = 16
    grid = (bsz // tbatch,)
    flat = c2 * h2 * w2s

    def cspec(a):
        nd = a.ndim
        return pl.BlockSpec(tuple(a.shape), lambda b, _n=nd: (0,) * _n)

    enc_consts = (w1c, b1m, w2c, b2m, wenc_h, benc, wdec_h, bdec_h)
    z, d_flat = pl.pallas_call(
        functools.partial(_enc_kernel, n_keep=latent - p),
        out_shape=(jax.ShapeDtypeStruct((bsz, latent), jnp.float32),
                   jax.ShapeDtypeStruct((bsz, flat), jnp.float32)),
        grid=grid,
        in_specs=[pl.BlockSpec((tbatch, cin, h, w), lambda b: (b, 0, 0, 0)),
                  pl.BlockSpec((tbatch, p), lambda b: (b, 0))]
                 + [cspec(a) for a in enc_consts],
        out_specs=(pl.BlockSpec((tbatch, latent), lambda b: (b, 0)),
                   pl.BlockSpec((tbatch, flat), lambda b: (b, 0))),
        compiler_params=pltpu.CompilerParams(
            dimension_semantics=("parallel",)),
    )(x, p_x, *enc_consts)

    d4 = d_flat.reshape(bsz, h2, w2s, c2)
    dec_consts = (w3ee, w3eo, w3oe, w3oo, b3m, w4ee, w4eo, w4oe, w4oo, b4m)
    x_rec = pl.pallas_call(
        _dec_kernel,
        out_shape=jax.ShapeDtypeStruct((bsz, cin, h, w), jnp.float32),
        grid=grid,
        in_specs=[pl.BlockSpec((tbatch, h2, w2s, c2), lambda b: (b, 0, 0, 0))]
                 + [cspec(a) for a in dec_consts],
        out_specs=pl.BlockSpec((tbatch, cin, h, w), lambda b: (b, 0, 0, 0)),
        compiler_params=pltpu.CompilerParams(
            dimension_semantics=("parallel",)),
    )(d4, *dec_consts)
    return x_rec, z


# speedup vs baseline: 2.1167x; 1.1728x over previous
import functools

import jax
import jax.numpy as jnp
from jax.experimental import pallas as pl
from jax.experimental.pallas import tpu as pltpu

_NEG_SLOPE = 0.01


def _lrelu(v):
    return jnp.where(v > 0, v, _NEG_SLOPE * v)


def _planes(x):
    tb, h, w, c = x.shape
    xr = x.reshape(tb, h // 2, 2, w, c)
    xe, xo = xr[:, :, 0], xr[:, :, 1]

    def cols(a):
        ar = a.reshape(tb, h // 2, w // 2, 2, c)
        return ar[:, :, :, 0], ar[:, :, :, 1]

    xee, xeo = cols(xe)
    xoe, xoo = cols(xo)
    return xee, xeo, xoe, xoo


def _shift_r(a):
    return jnp.concatenate([jnp.zeros_like(a[:, :1]), a[:, :-1]], axis=1)


def _shift_c(a):
    return jnp.concatenate([jnp.zeros_like(a[:, :, :1]), a[:, :, :-1]], axis=2)


def _conv_s2(x, wcol, bmap):
    tb, h, w, c = x.shape
    ho, wo = h // 2, w // 2
    xee, xeo, xoe, xoo = _planes(x)
    taps = [
        _shift_r(_shift_c(xoo)), _shift_r(xoe), _shift_r(xoo),
        _shift_c(xeo),           xee,           xeo,
        _shift_c(xoo),           xoe,           xoo,
    ]
    t = jnp.concatenate(taps, axis=3).reshape(tb * ho * wo, 9 * c)
    y = jnp.dot(t, wcol, preferred_element_type=jnp.float32)
    return y.reshape(tb, ho, wo, wcol.shape[1]) + bmap[None]


def _deconv_s2(d, wee, weo, woe, woo, bmap):
    tb, h, w, c = d.shape
    cout = wee.shape[1]
    dr = jnp.concatenate([d[:, 1:], jnp.zeros_like(d[:, :1])], axis=1)
    dc = jnp.concatenate([d[:, :, 1:], jnp.zeros_like(d[:, :, :1])], axis=2)
    drc = jnp.concatenate([dr[:, :, 1:], jnp.zeros_like(dr[:, :, :1])], axis=2)

    def mm(t, wc):
        k = t.shape[-1]
        y = jnp.dot(t.reshape(tb * h * w, k), wc,
                    preferred_element_type=jnp.float32)
        return y.reshape(tb, h, w, cout)

    y_ee = mm(d, wee)
    y_eo = mm(jnp.concatenate([dc, d], axis=3), weo)
    y_oe = mm(jnp.concatenate([dr, d], axis=3), woe)
    y_oo = mm(jnp.concatenate([drc, dr, dc, d], axis=3), woo)

    ye = jnp.stack([y_ee, y_eo], axis=3).reshape(tb, h, 2 * w, cout)
    yo = jnp.stack([y_oe, y_oo], axis=3).reshape(tb, h, 2 * w, cout)
    y = jnp.stack([ye, yo], axis=2).reshape(tb, 2 * h, 2 * w, cout)
    return y + bmap[None]


def _enc_kernel(x_ref, px_ref, w1_ref, b1_ref, w2_ref, b2_ref,
                wenc_ref, benc_ref, wdec_ref, bdec_ref,
                z_ref, d_ref, *, n_keep):
    tb = x_ref.shape[0]
    x = x_ref[...]
    h1 = _lrelu(_conv_s2(x, w1_ref[...], b1_ref[...]))
    h2 = _conv_s2(h1, w2_ref[...], b2_ref[...])
    e = h2.reshape(tb, -1)
    z0 = jnp.dot(e, wenc_ref[...], preferred_element_type=jnp.float32)
    z0 = z0 + benc_ref[...]
    z = jnp.concatenate([z0[:, :n_keep], px_ref[...]], axis=1)
    z_ref[...] = z
    d = jnp.dot(z, wdec_ref[...], preferred_element_type=jnp.float32)
    d_ref[...] = d + bdec_ref[...]


def _dec_kernel(d_ref,
                w3ee_ref, w3eo_ref, w3oe_ref, w3oo_ref, b3_ref,
                w4ee_ref, w4eo_ref, w4oe_ref, w4oo_ref, b4_ref,
                out_ref):
    d = d_ref[...]
    u1 = _lrelu(_deconv_s2(d, w3ee_ref[...], w3eo_ref[...], w3oe_ref[...],
                           w3oo_ref[...], b3_ref[...]))
    u2 = _deconv_s2(u1, w4ee_ref[...], w4eo_ref[...], w4oe_ref[...],
                    w4oo_ref[...], b4_ref[...])
    out_ref[...] = u2


def _conv_wcol(w):
    t, cout, cin = w.shape
    return jnp.transpose(w, (0, 2, 1)).reshape(t * cin, cout)


def _deconv_plane_w(w):
    wt = jnp.transpose(w, (0, 2, 1))
    wee = wt[4]
    weo = jnp.concatenate([wt[3], wt[5]], axis=0)
    woe = jnp.concatenate([wt[1], wt[7]], axis=0)
    woo = jnp.concatenate([wt[0], wt[2], wt[6], wt[8]], axis=0)
    return wee, weo, woe, woo


def _bias_map(b, ho, wo):
    return jnp.transpose(b, (1, 0)).reshape(ho, wo, b.shape[0])


def kernel(x, p_x, w1, b1, g1, w2, b2, g2, w3, b3, g3, w4, b4, g4,
           wenc, benc, wdec, bdec):
    del g1, g2, g3, g4
    bsz, cin, h, w = x.shape
    h1, w1s = h // 2, w // 2
    h2, w2s = h1 // 2, w1s // 2
    c1 = w1.shape[1]
    c2 = w2.shape[1]
    latent = wenc.shape[1]
    p = p_x.shape[1]

    w1c = _conv_wcol(w1)
    w2c = _conv_wcol(w2)
    w3ee, w3eo, w3oe, w3oo = _deconv_plane_w(w3)
    w4ee, w4eo, w4oe, w4oo = _deconv_plane_w(w4)
    b1m = _bias_map(b1, h1, w1s)
    b2m = _bias_map(b2, h2, w2s)
    b3m = _bias_map(b3, h1, w1s)
    b4m = _bias_map(b4, h, w)
    wenc_h = wenc.reshape(c2, h2, w2s, latent).transpose(1, 2, 0, 3)
    wenc_h = wenc_h.reshape(c2 * h2 * w2s, latent)
    wdec_h = wdec.reshape(latent, c2, h2, w2s).transpose(0, 2, 3, 1)
    wdec_h = wdec_h.reshape(latent, c2 * h2 * w2s)
    bdec_h = bdec.reshape(c2, h2, w2s).transpose(1, 2, 0).reshape(1, -1)

    tbatch = 16
    grid = (bsz // tbatch,)
    flat = c2 * h2 * w2s

    def cspec(a):
        nd = a.ndim
        return pl.BlockSpec(tuple(a.shape), lambda b, _n=nd: (0,) * _n)

    enc_consts = (w1c, b1m, w2c, b2m, wenc_h, benc, wdec_h, bdec_h)
    z, d_flat = pl.pallas_call(
        functools.partial(_enc_kernel, n_keep=latent - p),
        out_shape=(jax.ShapeDtypeStruct((bsz, latent), jnp.float32),
                   jax.ShapeDtypeStruct((bsz, flat), jnp.float32)),
        grid=grid,
        in_specs=[pl.BlockSpec((tbatch, h, w, cin), lambda b: (b, 0, 0, 0)),
                  pl.BlockSpec((tbatch, p), lambda b: (b, 0))]
                 + [cspec(a) for a in enc_consts],
        out_specs=(pl.BlockSpec((tbatch, latent), lambda b: (b, 0)),
                   pl.BlockSpec((tbatch, flat), lambda b: (b, 0))),
        compiler_params=pltpu.CompilerParams(
            dimension_semantics=("parallel",)),
    )(jnp.transpose(x, (0, 2, 3, 1)), p_x, *enc_consts)

    d4 = d_flat.reshape(bsz, h2, w2s, c2)
    dec_consts = (w3ee, w3eo, w3oe, w3oo, b3m, w4ee, w4eo, w4oe, w4oo, b4m)
    rec_nhwc = pl.pallas_call(
        _dec_kernel,
        out_shape=jax.ShapeDtypeStruct((bsz, h, w, cin), jnp.float32),
        grid=grid,
        in_specs=[pl.BlockSpec((tbatch, h2, w2s, c2), lambda b: (b, 0, 0, 0))]
                 + [cspec(a) for a in dec_consts],
        out_specs=pl.BlockSpec((tbatch, h, w, cin), lambda b: (b, 0, 0, 0)),
        compiler_params=pltpu.CompilerParams(
            dimension_semantics=("parallel",)),
    )(d4, *dec_consts)
    return jnp.transpose(rec_nhwc, (0, 3, 1, 2)), z
